# Initial kernel scaffold; baseline (speedup 1.0000x reference)
import functools

import jax
import jax.numpy as jnp
from jax.experimental import pallas as pl
from jax.experimental.pallas import tpu as pltpu
from jax.experimental.pallas import tpu_sc as plsc

VOCAB = 100000
DIM = 128
TOTAL_TOKENS = 32768
BATCH = 16

NUM_WORKERS = 32
TOK_PER_W = TOTAL_TOKENS // NUM_WORKERS
CHUNK = 128
NCHUNK = TOK_PER_W // CHUNK
LANES = 16
NVEC = DIM // LANES


def _sc_kernel(table_hbm, idx_hbm, seg_hbm, out_hbm, idx_v, seg_v, rows_v,
               acc_v):
  core = jax.lax.axis_index("c")
  sub = jax.lax.axis_index("s")
  wid = sub * 2 + core

  pltpu.sync_copy(idx_hbm.at[pl.ds(wid * NCHUNK, NCHUNK)], idx_v)
  pltpu.sync_copy(seg_hbm.at[pl.ds(wid * TOK_PER_W, TOK_PER_W)], seg_v)

  zero = jnp.zeros((LANES,), jnp.float32)

  @pl.loop(0, BATCH)
  def _(r):
    for v in range(NVEC):
      acc_v[r, pl.ds(v * LANES, LANES)] = zero

  for c in range(NCHUNK):
    pltpu.sync_copy(table_hbm.at[idx_v.at[c]], rows_v)

    @pl.loop(0, CHUNK)
    def _(j):
      s = seg_v[c * CHUNK + j]
      for v in range(NVEC):
        plsc.addupdate(acc_v.at[s, pl.ds(v * LANES, LANES)],
                       rows_v[j, pl.ds(v * LANES, LANES)])

  pltpu.sync_copy(acc_v, out_hbm.at[wid])


def kernel(table, flat_indices, segment_ids):
  idx2d = flat_indices.reshape(NUM_WORKERS * NCHUNK, CHUNK)
  mesh = plsc.VectorSubcoreMesh(core_axis_name="c", subcore_axis_name="s")
  run = pl.kernel(
      _sc_kernel,
      out_type=jax.ShapeDtypeStruct((NUM_WORKERS, BATCH, DIM), jnp.float32),
      mesh=mesh,
      scratch_types=[
          pltpu.VMEM((NCHUNK, CHUNK), jnp.int32),
          pltpu.VMEM((TOK_PER_W,), jnp.int32),
          pltpu.VMEM((CHUNK, DIM), jnp.float32),
          pltpu.VMEM((BATCH, DIM), jnp.float32),
      ],
  )
  partials = run(table, idx2d, segment_ids)
  return partials.sum(axis=0)


# speedup vs baseline: 2.2268x; 2.2268x over previous
import functools

import jax
import jax.numpy as jnp
from jax.experimental import pallas as pl
from jax.experimental.pallas import tpu as pltpu
from jax.experimental.pallas import tpu_sc as plsc

VOCAB = 100000
DIM = 128
TOTAL_TOKENS = 32768
BATCH = 16

NUM_WORKERS = 32
TOK_PER_W = TOTAL_TOKENS // NUM_WORKERS
CHUNK = 128
NCHUNK = TOK_PER_W // CHUNK
LANES = 16
NVEC = DIM // LANES


def _sc_kernel(table_hbm, idx_hbm, seg_hbm, out_hbm, idx_v, seg_v, rows_v,
               acc_v):
  core = jax.lax.axis_index("c")
  sub = jax.lax.axis_index("s")
  wid = sub * 2 + core

  pltpu.sync_copy(idx_hbm.at[pl.ds(wid * NCHUNK, NCHUNK)], idx_v)
  pltpu.sync_copy(seg_hbm.at[pl.ds(wid * TOK_PER_W, TOK_PER_W)], seg_v)

  zero = jnp.zeros((LANES,), jnp.float32)

  @pl.loop(0, BATCH)
  def _(r):
    for v in range(NVEC):
      acc_v[r, pl.ds(v * LANES, LANES)] = zero

  for c in range(NCHUNK):
    pltpu.sync_copy(table_hbm.at[idx_v.at[c]], rows_v)

    @pl.loop(0, CHUNK // LANES)
    def _(g):
      segvec = seg_v[pl.ds(c * CHUNK + g * LANES, LANES)]
      for k in range(LANES):
        s = segvec[k]
        for v in range(NVEC):
          plsc.addupdate(acc_v.at[s, pl.ds(v * LANES, LANES)],
                         rows_v[g * LANES + k, pl.ds(v * LANES, LANES)])

  pltpu.sync_copy(acc_v, out_hbm.at[wid])


def kernel(table, flat_indices, segment_ids):
  idx2d = flat_indices.reshape(NUM_WORKERS * NCHUNK, CHUNK)
  mesh = plsc.VectorSubcoreMesh(core_axis_name="c", subcore_axis_name="s")
  run = pl.kernel(
      _sc_kernel,
      out_type=jax.ShapeDtypeStruct((NUM_WORKERS, BATCH, DIM), jnp.float32),
      mesh=mesh,
      scratch_types=[
          pltpu.VMEM((NCHUNK, CHUNK), jnp.int32),
          pltpu.VMEM((TOK_PER_W,), jnp.int32),
          pltpu.VMEM((CHUNK, DIM), jnp.float32),
          pltpu.VMEM((BATCH, DIM), jnp.float32),
      ],
  )
  partials = run(table, idx2d, segment_ids)
  return partials.sum(axis=0)


# speedup vs baseline: 3.8423x; 1.7255x over previous
import functools

import jax
import jax.numpy as jnp
from jax.experimental import pallas as pl
from jax.experimental.pallas import tpu as pltpu
from jax.experimental.pallas import tpu_sc as plsc

VOCAB = 100000
DIM = 128
TOTAL_TOKENS = 32768
BATCH = 16

NUM_WORKERS = 32
TOK_PER_W = TOTAL_TOKENS // NUM_WORKERS
CHUNK = 128
NCHUNK = TOK_PER_W // CHUNK
LANES = 16
NVEC = DIM // LANES


def _sc_kernel(table_hbm, idx_hbm, seg_hbm, out_hbm, idx_v, seg_v, rows_v,
               acc_v, acc_sh):
  core = jax.lax.axis_index("c")
  sub = jax.lax.axis_index("s")
  wid = sub * 2 + core

  pltpu.sync_copy(idx_hbm.at[pl.ds(wid * NCHUNK, NCHUNK)], idx_v)
  pltpu.sync_copy(seg_hbm.at[pl.ds(wid * NCHUNK, NCHUNK)], seg_v)

  zero = jnp.zeros((LANES,), jnp.float32)

  @pl.when(sub == 0)
  def _():
    @pl.loop(0, BATCH)
    def _(r):
      for v in range(NVEC):
        acc_v[r, pl.ds(v * LANES, LANES)] = zero

    pltpu.sync_copy(acc_v, acc_sh)

  plsc.subcore_barrier()

  for c in range(NCHUNK):
    pltpu.sync_copy(table_hbm.at[idx_v.at[c]], rows_v)
    pltpu.sync_copy(rows_v, acc_sh.at[seg_v.at[c]], add=True)

  plsc.subcore_barrier()

  @pl.when(sub == 0)
  def _():
    pltpu.sync_copy(acc_sh, out_hbm.at[core])


def kernel(table, flat_indices, segment_ids):
  idx2d = flat_indices.reshape(NUM_WORKERS * NCHUNK, CHUNK)
  seg2d = segment_ids.reshape(NUM_WORKERS * NCHUNK, CHUNK)
  mesh = plsc.VectorSubcoreMesh(core_axis_name="c", subcore_axis_name="s")
  run = pl.kernel(
      _sc_kernel,
      out_type=jax.ShapeDtypeStruct((2, BATCH, DIM), jnp.float32),
      mesh=mesh,
      scratch_types=[
          pltpu.VMEM((NCHUNK, CHUNK), jnp.int32),
          pltpu.VMEM((NCHUNK, CHUNK), jnp.int32),
          pltpu.VMEM((CHUNK, DIM), jnp.float32),
          pltpu.VMEM((BATCH, DIM), jnp.float32),
          pltpu.VMEM_SHARED((BATCH, DIM), jnp.float32),
      ],
  )
  partials = run(table, idx2d, seg2d)
  return partials.sum(axis=0)


# speedup vs baseline: 4.4864x; 1.1676x over previous
import jax
import jax.numpy as jnp
from jax import lax
from jax.experimental import pallas as pl
from jax.experimental.pallas import tpu as pltpu
from jax.experimental.pallas import tpu_sc as plsc

VOCAB = 100000
DIM = 128
TOTAL_TOKENS = 32768
BATCH = 16

NUM_WORKERS = 32
TOK_PER_W = TOTAL_TOKENS // NUM_WORKERS
CHUNK = 128
NCHUNK = TOK_PER_W // CHUNK
LANES = 16
NVEC = DIM // LANES


def _sc_kernel(table_hbm, idx_hbm, seg_hbm, out_hbm, idx_v, seg_v, rows_v,
               acc_v, iota_v, acc_sh, sems):
  core = jax.lax.axis_index("c")
  sub = jax.lax.axis_index("s")
  wid = sub * 2 + core

  pltpu.sync_copy(idx_hbm.at[pl.ds(wid * NCHUNK, NCHUNK)], idx_v)
  pltpu.sync_copy(seg_hbm.at[pl.ds(wid * NCHUNK, NCHUNK)], seg_v)

  zero = jnp.zeros((LANES,), jnp.float32)
  iota_v[...] = lax.broadcasted_iota(jnp.int32, (LANES,), 0)

  @pl.loop(0, BATCH)
  def _(r):
    for v in range(NVEC):
      acc_v[r, pl.ds(v * LANES, LANES)] = zero

  @pl.when(sub == 0)
  def _():
    pltpu.sync_copy(acc_v, acc_sh)

  plsc.subcore_barrier()

  copies = [None, None]
  copies[0] = pltpu.async_copy(table_hbm.at[idx_v.at[0]], rows_v.at[0],
                               sems.at[0])

  for c in range(NCHUNK):
    buf = c % 2
    if c + 1 < NCHUNK:
      copies[1 - buf] = pltpu.async_copy(
          table_hbm.at[idx_v.at[c + 1]], rows_v.at[1 - buf], sems.at[1 - buf])
    copies[buf].wait()

    first = seg_v[c, pl.ds(0, LANES)][0]
    last = seg_v[c, pl.ds(CHUNK - LANES, LANES)][LANES - 1]
    pure = first == last

    @pl.when(pure)
    def _(c=c, buf=buf, first=first):
      def body(j, carry):
        return tuple(
            carry[v] + rows_v[buf, j, pl.ds(v * LANES, LANES)]
            for v in range(NVEC))

      folded = lax.fori_loop(0, CHUNK, body, (zero,) * NVEC)
      for v in range(NVEC):
        plsc.addupdate(acc_v.at[first, pl.ds(v * LANES, LANES)], folded[v])

    @pl.when(jnp.logical_not(pure))
    def _(c=c, buf=buf):
      pltpu.sync_copy(rows_v.at[buf], acc_sh.at[seg_v.at[c]], add=True)

  pltpu.sync_copy(acc_v, acc_sh.at[iota_v], add=True)
  plsc.subcore_barrier()

  @pl.when(sub == 0)
  def _():
    pltpu.sync_copy(acc_sh, out_hbm.at[core])


def kernel(table, flat_indices, segment_ids):
  idx2d = flat_indices.reshape(NUM_WORKERS * NCHUNK, CHUNK)
  seg2d = segment_ids.reshape(NUM_WORKERS * NCHUNK, CHUNK)
  mesh = plsc.VectorSubcoreMesh(core_axis_name="c", subcore_axis_name="s")
  run = pl.kernel(
      _sc_kernel,
      out_type=jax.ShapeDtypeStruct((2, BATCH, DIM), jnp.float32),
      mesh=mesh,
      scratch_types=[
          pltpu.VMEM((NCHUNK, CHUNK), jnp.int32),
          pltpu.VMEM((NCHUNK, CHUNK), jnp.int32),
          pltpu.VMEM((2, CHUNK, DIM), jnp.float32),
          pltpu.VMEM((BATCH, DIM), jnp.float32),
          pltpu.VMEM((LANES,), jnp.int32),
          pltpu.VMEM_SHARED((BATCH, DIM), jnp.float32),
          pltpu.SemaphoreType.DMA((2,)),
      ],
  )
  partials = run(table, idx2d, seg2d)
  return partials.sum(axis=0)


# speedup vs baseline: 4.6023x; 1.0258x over previous
import jax
import jax.numpy as jnp
from jax import lax
from jax.experimental import pallas as pl
from jax.experimental.pallas import tpu as pltpu
from jax.experimental.pallas import tpu_sc as plsc

VOCAB = 100000
DIM = 128
TOTAL_TOKENS = 32768
BATCH = 16

NUM_WORKERS = 32
TOK_PER_W = TOTAL_TOKENS // NUM_WORKERS
CHUNK = 128
NCHUNK = TOK_PER_W // CHUNK
LANES = 16
NVEC = DIM // LANES


def _sc_kernel(table_hbm, idx_hbm, seg_hbm, out_hbm, idx_v, seg_v, rows_v,
               acc_v, iota_v, acc_sh, sems):
  core = jax.lax.axis_index("c")
  sub = jax.lax.axis_index("s")
  wid = sub * 2 + core

  pltpu.sync_copy(idx_hbm.at[pl.ds(wid * NCHUNK, NCHUNK)], idx_v)
  pltpu.sync_copy(seg_hbm.at[pl.ds(wid * NCHUNK, NCHUNK)], seg_v)

  zero = jnp.zeros((LANES,), jnp.float32)
  iota_v[...] = lax.broadcasted_iota(jnp.int32, (LANES,), 0)

  @pl.loop(0, BATCH)
  def _(r):
    for v in range(NVEC):
      acc_v[r, pl.ds(v * LANES, LANES)] = zero

  @pl.when(sub == 0)
  def _():
    pltpu.sync_copy(acc_v, acc_sh)

  plsc.subcore_barrier()

  NBUF = 4
  copies = [None] * NBUF
  for c in range(NBUF - 1):
    copies[c] = pltpu.async_copy(table_hbm.at[idx_v.at[c]], rows_v.at[c],
                                 sems.at[c])

  for c in range(NCHUNK):
    buf = c % NBUF
    nxt = c + NBUF - 1
    if nxt < NCHUNK:
      copies[nxt % NBUF] = pltpu.async_copy(
          table_hbm.at[idx_v.at[nxt]], rows_v.at[nxt % NBUF],
          sems.at[nxt % NBUF])
    copies[buf].wait()

    first = seg_v[c, pl.ds(0, LANES)][0]
    last = seg_v[c, pl.ds(CHUNK - LANES, LANES)][LANES - 1]
    pure = first == last

    @pl.when(pure)
    def _(c=c, buf=buf, first=first):
      def body(j, carry):
        j2 = j * 2
        return tuple(
            carry[v] + (rows_v[buf, j2, pl.ds(v * LANES, LANES)] +
                        rows_v[buf, j2 + 1, pl.ds(v * LANES, LANES)])
            for v in range(NVEC))

      folded = lax.fori_loop(0, CHUNK // 2, body, (zero,) * NVEC)
      for v in range(NVEC):
        plsc.addupdate(acc_v.at[first, pl.ds(v * LANES, LANES)], folded[v])

    @pl.when(jnp.logical_not(pure))
    def _(c=c, buf=buf):
      pltpu.sync_copy(rows_v.at[buf], acc_sh.at[seg_v.at[c]], add=True)

  pltpu.sync_copy(acc_v, acc_sh.at[iota_v], add=True)
  plsc.subcore_barrier()

  @pl.when(sub == 0)
  def _():
    pltpu.sync_copy(acc_sh, out_hbm.at[core])


def kernel(table, flat_indices, segment_ids):
  idx2d = flat_indices.reshape(NUM_WORKERS * NCHUNK, CHUNK)
  seg2d = segment_ids.reshape(NUM_WORKERS * NCHUNK, CHUNK)
  mesh = plsc.VectorSubcoreMesh(core_axis_name="c", subcore_axis_name="s")
  run = pl.kernel(
      _sc_kernel,
      out_type=jax.ShapeDtypeStruct((2, BATCH, DIM), jnp.float32),
      mesh=mesh,
      scratch_types=[
          pltpu.VMEM((NCHUNK, CHUNK), jnp.int32),
          pltpu.VMEM((NCHUNK, CHUNK), jnp.int32),
          pltpu.VMEM((4, CHUNK, DIM), jnp.float32),
          pltpu.VMEM((BATCH, DIM), jnp.float32),
          pltpu.VMEM((LANES,), jnp.int32),
          pltpu.VMEM_SHARED((BATCH, DIM), jnp.float32),
          pltpu.SemaphoreType.DMA((4,)),
      ],
  )
  partials = run(table, idx2d, seg2d)
  return partials.sum(axis=0)


# speedup vs baseline: 4.6572x; 1.0119x over previous
import jax
import jax.numpy as jnp
from jax import lax
from jax.experimental import pallas as pl
from jax.experimental.pallas import tpu as pltpu
from jax.experimental.pallas import tpu_sc as plsc

VOCAB = 100000
DIM = 128
TOTAL_TOKENS = 32768
BATCH = 16

NUM_WORKERS = 32
TOK_PER_W = TOTAL_TOKENS // NUM_WORKERS
CHUNK = 64
NCHUNK = TOK_PER_W // CHUNK
NBUF = 4
LANES = 16
NVEC = DIM // LANES


def _sc_kernel(table_hbm, idx_hbm, seg_hbm, out_hbm, idx_v, seg_v, rows_v,
               acc_v, iota_v, acc_sh, sems):
  core = jax.lax.axis_index("c")
  sub = jax.lax.axis_index("s")
  wid = sub * 2 + core

  pltpu.sync_copy(idx_hbm.at[pl.ds(wid * NCHUNK, NCHUNK)], idx_v)
  pltpu.sync_copy(seg_hbm.at[pl.ds(wid * NCHUNK, NCHUNK)], seg_v)

  zero = jnp.zeros((LANES,), jnp.float32)
  iota_v[...] = lax.broadcasted_iota(jnp.int32, (LANES,), 0)

  @pl.loop(0, BATCH)
  def _(r):
    for v in range(NVEC):
      acc_v[r, pl.ds(v * LANES, LANES)] = zero

  @pl.when(sub == 0)
  def _():
    pltpu.sync_copy(acc_v, acc_sh)

  plsc.subcore_barrier()

  for b in range(NBUF - 1):
    pltpu.async_copy(table_hbm.at[idx_v.at[b]], rows_v.at[b], sems.at[b])

  def chunk_body(c, buf):
    nxt = c + NBUF - 1
    @pl.when(nxt < NCHUNK)
    def _():
      pltpu.async_copy(table_hbm.at[idx_v.at[nxt]], rows_v.at[(NBUF - 1 + buf)
                                                              % NBUF],
                       sems.at[(NBUF - 1 + buf) % NBUF])
    pltpu.make_async_copy(table_hbm.at[idx_v.at[c]], rows_v.at[buf],
                          sems.at[buf]).wait()

    first = seg_v[c, pl.ds(0, LANES)][0]
    last = seg_v[c, pl.ds(CHUNK - LANES, LANES)][LANES - 1]
    pure = first == last

    @pl.when(pure)
    def _():
      def body(j, carry):
        j2 = j * 2
        return tuple(
            carry[v] + (rows_v[buf, j2, pl.ds(v * LANES, LANES)] +
                        rows_v[buf, j2 + 1, pl.ds(v * LANES, LANES)])
            for v in range(NVEC))

      folded = lax.fori_loop(0, CHUNK // 2, body, (zero,) * NVEC)
      for v in range(NVEC):
        plsc.addupdate(acc_v.at[first, pl.ds(v * LANES, LANES)], folded[v])

    @pl.when(jnp.logical_not(pure))
    def _():
      pltpu.sync_copy(rows_v.at[buf], acc_sh.at[seg_v.at[c]], add=True)

  @pl.loop(0, NCHUNK // NBUF)
  def _(i):
    for b in range(NBUF):
      chunk_body(i * NBUF + b, b)

  pltpu.sync_copy(acc_v, acc_sh.at[iota_v], add=True)
  plsc.subcore_barrier()

  @pl.when(sub == 0)
  def _():
    pltpu.sync_copy(acc_sh, out_hbm.at[core])


def kernel(table, flat_indices, segment_ids):
  idx2d = flat_indices.reshape(NUM_WORKERS * NCHUNK, CHUNK)
  seg2d = segment_ids.reshape(NUM_WORKERS * NCHUNK, CHUNK)
  mesh = plsc.VectorSubcoreMesh(core_axis_name="c", subcore_axis_name="s")
  run = pl.kernel(
      _sc_kernel,
      out_type=jax.ShapeDtypeStruct((2, BATCH, DIM), jnp.float32),
      mesh=mesh,
      scratch_types=[
          pltpu.VMEM((NCHUNK, CHUNK), jnp.int32),
          pltpu.VMEM((NCHUNK, CHUNK), jnp.int32),
          pltpu.VMEM((NBUF, CHUNK, DIM), jnp.float32),
          pltpu.VMEM((BATCH, DIM), jnp.float32),
          pltpu.VMEM((LANES,), jnp.int32),
          pltpu.VMEM_SHARED((BATCH, DIM), jnp.float32),
          pltpu.SemaphoreType.DMA((NBUF,)),
      ],
  )
  partials = run(table, idx2d, seg2d)
  return partials.sum(axis=0)
